# baseline (speedup 1.0000x reference)
import functools

import jax
import jax.numpy as jnp
from jax import lax
from jax.experimental import pallas as pl
from jax.experimental.pallas import tpu as pltpu
from jax.experimental.pallas import tpu_sc as plsc

EB = 64
NTILES = 16
NSHARD = 32
FC = 128
PH = 80
ACC_ROWS = 10240
ZR = ACC_ROWS // NTILES


def _agg_kernel(N, NB):
  mesh = plsc.VectorSubcoreMesh(core_axis_name="c", subcore_axis_name="s")

  @functools.partial(
      pl.kernel,
      out_type=jax.ShapeDtypeStruct((2 * ACC_ROWS, FC), jnp.float32),
      mesh=mesh,
      scratch_types=[
          pltpu.VMEM((PH, EB), jnp.int32),
          pltpu.VMEM((PH, EB), jnp.int32),
          pltpu.VMEM((2, EB, FC), jnp.float32),
          pltpu.VMEM_SHARED((ACC_ROWS, FC), jnp.float32),
          pltpu.SemaphoreType.DMA,
          pltpu.SemaphoreType.DMA,
      ],
  )
  def k(h, srcp, dstp, zeros, out, src_v, dst_v, rows_v, acc, s0, s1):
    c = lax.axis_index("c")
    s = lax.axis_index("s")
    w = s * 2 + c
    pltpu.sync_copy(zeros, acc.at[pl.ds(s * ZR, ZR)])
    plsc.subcore_barrier()

    def phase(p, carry):
      pltpu.sync_copy(srcp.at[w, pl.ds(p * PH, PH)], src_v)
      pltpu.sync_copy(dstp.at[w, pl.ds(p * PH, PH)], dst_v)

      def pair(q, inner):
        b0 = 2 * q
        b1 = 2 * q + 1
        d0 = pltpu.async_copy(h.at[src_v.at[b0]], rows_v.at[0], s0)
        d1 = pltpu.async_copy(h.at[src_v.at[b1]], rows_v.at[1], s1)
        d0.wait()
        pltpu.sync_copy(rows_v.at[0], acc.at[dst_v.at[b0]], add=True)
        d1.wait()
        pltpu.sync_copy(rows_v.at[1], acc.at[dst_v.at[b1]], add=True)
        return inner

      lax.fori_loop(0, PH // 2, pair, 0)
      return carry

    lax.fori_loop(0, NB // PH, phase, 0)
    plsc.subcore_barrier()
    pltpu.sync_copy(acc.at[pl.ds(s * ZR, ZR)],
                    out.at[pl.ds(c * ACC_ROWS + s * ZR, ZR)])

  return k


def _combine_call(aggp, h, degp, wl, bl, wr, wln):
  N, fin = h.shape
  fout = wl.shape[1]
  has_ln = wln is not None
  blk = 400
  grid = (N // blk,)

  def body(agg_ref, h_ref, deg_ref, wl_ref, bl_ref, wr_ref, *rest):
    if has_ln:
      wln_ref, out_ref = rest
    else:
      (out_ref,) = rest
    d = deg_ref[0, :, 0:1] + deg_ref[1, :, 0:1]
    inv = 1.0 / jnp.maximum(d, 1.0)
    a = (agg_ref[0] + agg_ref[1]) * inv
    hv = h_ref[...]
    y = (jnp.dot(a, wl_ref[...], preferred_element_type=jnp.float32)
         + bl_ref[...]
         + jnp.dot(hv, wr_ref[...], preferred_element_type=jnp.float32))
    y = jnp.maximum(y, 0.0)
    if has_ln:
      y = jnp.maximum(
          jnp.dot(y, wln_ref[...], preferred_element_type=jnp.float32) + hv,
          0.0)
    out_ref[...] = y

  in_specs = [
      pl.BlockSpec((2, blk, fin), lambda i: (0, i, 0)),
      pl.BlockSpec((blk, fin), lambda i: (i, 0)),
      pl.BlockSpec((2, blk, 128), lambda i: (0, i, 0)),
      pl.BlockSpec((fin, fout), lambda i: (0, 0)),
      pl.BlockSpec((1, fout), lambda i: (0, 0)),
      pl.BlockSpec((fin, fout), lambda i: (0, 0)),
  ]
  args = [aggp, h, degp, wl, bl.reshape(1, fout), wr]
  if has_ln:
    in_specs.append(pl.BlockSpec((fout, fout), lambda i: (0, 0)))
    args.append(wln)

  return pl.pallas_call(
      body,
      grid=grid,
      in_specs=in_specs,
      out_specs=pl.BlockSpec((blk, fout), lambda i: (i, 0)),
      out_shape=jax.ShapeDtypeStruct((N, fout), jnp.float32),
  )(*args)


def _pool_mlp_call(h, batch, wg1, bg1, wg2, bg2, wg3, bg3, B):
  N, F = h.shape
  CH = 64

  def body(h_ref, b_ref, w1_ref, b1_ref, w2_ref, b2_ref, w3_ref, b3_ref,
           out_ref, pooled_ref):
    brow = b_ref[0, :]

    def graph_body(g, carry):
      start = jnp.sum((brow < g).astype(jnp.int32))
      end = jnp.sum((brow <= g).astype(jnp.int32))

      def cond(c):
        k, _ = c
        return k * CH < end

      def chunk(c):
        k, acc = c
        base = pl.multiple_of(jnp.minimum(k * CH, N - CH), 8)
        rows = h_ref[pl.ds(base, CH), :]
        ridx = base + lax.broadcasted_iota(jnp.int32, (CH, 1), 0)
        m = (ridx >= start) & (ridx < end)
        rows = jnp.where(m, rows, -jnp.inf)
        acc = jnp.maximum(acc, jnp.max(rows, axis=0, keepdims=True))
        return k + 1, acc

      _, acc = lax.while_loop(cond, chunk,
                              (start // CH,
                               jnp.full((1, F), -jnp.inf, jnp.float32)))
      acc = jnp.where(jnp.isfinite(acc), acc, 0.0)
      pooled_ref[pl.ds(g, 1), :] = acc
      return carry

    lax.fori_loop(0, B, graph_body, 0)
    p = pooled_ref[...]
    g1 = jnp.maximum(
        jnp.dot(p, w1_ref[...], preferred_element_type=jnp.float32)
        + b1_ref[...], 0.0)
    g2 = jnp.dot(g1, w2_ref[...], preferred_element_type=jnp.float32) + b2_ref[...]
    out_ref[...] = jnp.maximum(
        jnp.dot(g2, w3_ref[...], preferred_element_type=jnp.float32)
        + b3_ref[...], 0.0)

  return pl.pallas_call(
      body,
      out_shape=jax.ShapeDtypeStruct((B, wg3.shape[1]), jnp.float32),
      scratch_shapes=[pltpu.VMEM((B, F), jnp.float32)],
  )(h, batch.reshape(1, N), wg1, bg1.reshape(1, -1), wg2, bg2.reshape(1, -1),
    wg3, bg3.reshape(1, -1))


def kernel(x, edge_index, batch, Wl1, bl1, Wr1, Wl2, bl2, Wr2,
           Wls, bls, Wrs, Wln, Wg1, bg1, Wg2, bg2, Wg3, bg3):
  N, _ = x.shape
  E = edge_index.shape[1]
  B = 256
  src = edge_index[0]
  dst = edge_index[1]

  NB = PH * (-(-E // (NSHARD * EB * PH)))
  EP = NSHARD * NB * EB
  srcp = jnp.concatenate(
      [src, jnp.zeros((EP - E,), jnp.int32)]).reshape(NSHARD, NB, EB)
  pad_dst = N + jnp.arange(EP - E, dtype=jnp.int32) % (ACC_ROWS - N)
  dstp = jnp.concatenate([dst, pad_dst]).reshape(NSHARD, NB, EB)
  zeros = jnp.zeros((ZR, FC), jnp.float32)

  aggk = _agg_kernel(N, NB)

  def parts(hcols):
    out = aggk(hcols, srcp, dstp, zeros)
    return out.reshape(2, ACC_ROWS, FC)[:, :N, :]

  def aggregate(h):
    if h.shape[1] == FC:
      return parts(h)
    halves = [parts(h[:, i * FC:(i + 1) * FC]) for i in range(h.shape[1] // FC)]
    return jnp.concatenate(halves, axis=2)

  degp = parts(jnp.ones((N, FC), jnp.float32))

  h = _combine_call(aggregate(x), x, degp, Wl1, bl1, Wr1, None)
  h = _combine_call(aggregate(h), h, degp, Wl2, bl2, Wr2, None)
  for _ in range(4):
    h = _combine_call(aggregate(h), h, degp, Wls, bls, Wrs, Wln)

  return _pool_mlp_call(h, batch, Wg1, bg1, Wg2, bg2, Wg3, bg3, B)


# speedup vs baseline: 1.0855x; 1.0855x over previous
import functools

import jax
import jax.numpy as jnp
from jax import lax
from jax.experimental import pallas as pl
from jax.experimental.pallas import tpu as pltpu
from jax.experimental.pallas import tpu_sc as plsc

EB = 128
NTILES = 16
NSHARD = 32
FC = 128
PH = 8
ACC_ROWS = 10240
ZR = ACC_ROWS // NTILES


def _agg_kernel(N, NB):
  mesh = plsc.VectorSubcoreMesh(core_axis_name="c", subcore_axis_name="s")

  @functools.partial(
      pl.kernel,
      out_type=jax.ShapeDtypeStruct((2 * ACC_ROWS, FC), jnp.float32),
      mesh=mesh,
      scratch_types=[
          pltpu.VMEM((PH, EB), jnp.int32),
          pltpu.VMEM((PH, EB), jnp.int32),
          pltpu.VMEM((2, EB, FC), jnp.float32),
          pltpu.VMEM_SHARED((ACC_ROWS, FC), jnp.float32),
          pltpu.SemaphoreType.DMA,
          pltpu.SemaphoreType.DMA,
      ],
  )
  def k(h, srcp, dstp, zeros, out, src_v, dst_v, rows_v, acc, s0, s1):
    c = lax.axis_index("c")
    s = lax.axis_index("s")
    w = s * 2 + c
    pltpu.sync_copy(zeros, acc.at[pl.ds(s * ZR, ZR)])
    plsc.subcore_barrier()

    def phase(p, carry):
      pltpu.sync_copy(srcp.at[w, pl.ds(p * PH, PH)], src_v)
      pltpu.sync_copy(dstp.at[w, pl.ds(p * PH, PH)], dst_v)

      def pair(q, inner):
        b0 = 2 * q
        b1 = 2 * q + 1
        d0 = pltpu.async_copy(h.at[src_v.at[b0]], rows_v.at[0], s0)
        d1 = pltpu.async_copy(h.at[src_v.at[b1]], rows_v.at[1], s1)
        d0.wait()
        pltpu.sync_copy(rows_v.at[0], acc.at[dst_v.at[b0]], add=True)
        d1.wait()
        pltpu.sync_copy(rows_v.at[1], acc.at[dst_v.at[b1]], add=True)
        return inner

      lax.fori_loop(0, PH // 2, pair, 0)
      return carry

    lax.fori_loop(0, NB // PH, phase, 0)
    plsc.subcore_barrier()
    pltpu.sync_copy(acc.at[pl.ds(s * ZR, ZR)],
                    out.at[pl.ds(c * ACC_ROWS + s * ZR, ZR)])

  return k


def _combine_call(aggp, h, degp, wl, bl, wr, wln):
  N, fin = h.shape
  fout = wl.shape[1]
  has_ln = wln is not None
  blk = 400
  grid = (N // blk,)

  def body(agg_ref, h_ref, deg_ref, wl_ref, bl_ref, wr_ref, *rest):
    if has_ln:
      wln_ref, out_ref = rest
    else:
      (out_ref,) = rest
    d = deg_ref[0, :, 0:1] + deg_ref[1, :, 0:1]
    inv = 1.0 / jnp.maximum(d, 1.0)
    a = (agg_ref[0] + agg_ref[1]) * inv
    hv = h_ref[...]
    y = (jnp.dot(a, wl_ref[...], preferred_element_type=jnp.float32)
         + bl_ref[...]
         + jnp.dot(hv, wr_ref[...], preferred_element_type=jnp.float32))
    y = jnp.maximum(y, 0.0)
    if has_ln:
      y = jnp.maximum(
          jnp.dot(y, wln_ref[...], preferred_element_type=jnp.float32) + hv,
          0.0)
    out_ref[...] = y

  in_specs = [
      pl.BlockSpec((2, blk, fin), lambda i: (0, i, 0)),
      pl.BlockSpec((blk, fin), lambda i: (i, 0)),
      pl.BlockSpec((2, blk, 128), lambda i: (0, i, 0)),
      pl.BlockSpec((fin, fout), lambda i: (0, 0)),
      pl.BlockSpec((1, fout), lambda i: (0, 0)),
      pl.BlockSpec((fin, fout), lambda i: (0, 0)),
  ]
  args = [aggp, h, degp, wl, bl.reshape(1, fout), wr]
  if has_ln:
    in_specs.append(pl.BlockSpec((fout, fout), lambda i: (0, 0)))
    args.append(wln)

  return pl.pallas_call(
      body,
      grid=grid,
      in_specs=in_specs,
      out_specs=pl.BlockSpec((blk, fout), lambda i: (i, 0)),
      out_shape=jax.ShapeDtypeStruct((N, fout), jnp.float32),
  )(*args)


def _pool_mlp_call(h, batch, wg1, bg1, wg2, bg2, wg3, bg3, B):
  N, F = h.shape
  CH = 64

  def body(h_ref, b_ref, w1_ref, b1_ref, w2_ref, b2_ref, w3_ref, b3_ref,
           out_ref, pooled_ref):
    brow = b_ref[0, :]

    def graph_body(g, carry):
      start = jnp.sum((brow < g).astype(jnp.int32))
      end = jnp.sum((brow <= g).astype(jnp.int32))

      def cond(c):
        k, _ = c
        return k * CH < end

      def chunk(c):
        k, acc = c
        base = pl.multiple_of(jnp.minimum(k * CH, N - CH), 8)
        rows = h_ref[pl.ds(base, CH), :]
        ridx = base + lax.broadcasted_iota(jnp.int32, (CH, 1), 0)
        m = (ridx >= start) & (ridx < end)
        rows = jnp.where(m, rows, -jnp.inf)
        acc = jnp.maximum(acc, jnp.max(rows, axis=0, keepdims=True))
        return k + 1, acc

      _, acc = lax.while_loop(cond, chunk,
                              (start // CH,
                               jnp.full((1, F), -jnp.inf, jnp.float32)))
      acc = jnp.where(jnp.isfinite(acc), acc, 0.0)
      pooled_ref[pl.ds(g, 1), :] = acc
      return carry

    lax.fori_loop(0, B, graph_body, 0)
    p = pooled_ref[...]
    g1 = jnp.maximum(
        jnp.dot(p, w1_ref[...], preferred_element_type=jnp.float32)
        + b1_ref[...], 0.0)
    g2 = jnp.dot(g1, w2_ref[...], preferred_element_type=jnp.float32) + b2_ref[...]
    out_ref[...] = jnp.maximum(
        jnp.dot(g2, w3_ref[...], preferred_element_type=jnp.float32)
        + b3_ref[...], 0.0)

  return pl.pallas_call(
      body,
      out_shape=jax.ShapeDtypeStruct((B, wg3.shape[1]), jnp.float32),
      scratch_shapes=[pltpu.VMEM((B, F), jnp.float32)],
  )(h, batch.reshape(1, N), wg1, bg1.reshape(1, -1), wg2, bg2.reshape(1, -1),
    wg3, bg3.reshape(1, -1))


def kernel(x, edge_index, batch, Wl1, bl1, Wr1, Wl2, bl2, Wr2,
           Wls, bls, Wrs, Wln, Wg1, bg1, Wg2, bg2, Wg3, bg3):
  N, _ = x.shape
  E = edge_index.shape[1]
  B = 256
  src = edge_index[0]
  dst = edge_index[1]

  NB = PH * (-(-E // (NSHARD * EB * PH)))
  EP = NSHARD * NB * EB
  srcp = jnp.concatenate(
      [src, jnp.zeros((EP - E,), jnp.int32)]).reshape(NSHARD, NB, EB)
  pad_dst = N + jnp.arange(EP - E, dtype=jnp.int32) % (ACC_ROWS - N)
  dstp = jnp.concatenate([dst, pad_dst]).reshape(NSHARD, NB, EB)
  zeros = jnp.zeros((ZR, FC), jnp.float32)

  aggk = _agg_kernel(N, NB)

  def parts(hcols):
    out = aggk(hcols, srcp, dstp, zeros)
    return out.reshape(2, ACC_ROWS, FC)[:, :N, :]

  def aggregate(h):
    if h.shape[1] == FC:
      return parts(h)
    halves = [parts(h[:, i * FC:(i + 1) * FC]) for i in range(h.shape[1] // FC)]
    return jnp.concatenate(halves, axis=2)

  degp = parts(jnp.ones((N, FC), jnp.float32))

  h = _combine_call(aggregate(x), x, degp, Wl1, bl1, Wr1, None)
  h = _combine_call(aggregate(h), h, degp, Wl2, bl2, Wr2, None)
  for _ in range(4):
    h = _combine_call(aggregate(h), h, degp, Wls, bls, Wrs, Wln)

  return _pool_mlp_call(h, batch, Wg1, bg1, Wg2, bg2, Wg3, bg3, B)


# speedup vs baseline: 2.8436x; 2.6196x over previous
import functools

import jax
import jax.numpy as jnp
from jax import lax
from jax.experimental import pallas as pl
from jax.experimental.pallas import tpu as pltpu
from jax.experimental.pallas import tpu_sc as plsc

EB = 128
NTILES = 16
NSHARD = 32
FC = 128
PH = 8
ACC_ROWS = 10240
ZR = ACC_ROWS // NTILES


def _agg_kernel(N, NB):
  mesh = plsc.VectorSubcoreMesh(core_axis_name="c", subcore_axis_name="s")

  @functools.partial(
      pl.kernel,
      out_type=jax.ShapeDtypeStruct((2 * ACC_ROWS, FC), jnp.float32),
      mesh=mesh,
      scratch_types=[
          pltpu.VMEM((NB, EB), jnp.int32),
          pltpu.VMEM((NB, EB), jnp.int32),
          pltpu.VMEM((EB, FC), jnp.float32),
          pltpu.VMEM_SHARED((ACC_ROWS, FC), jnp.float32),
      ],
  )
  def k(h, srcp, dstp, zeros, out, src_v, dst_v, rows_v, acc):
    c = lax.axis_index("c")
    s = lax.axis_index("s")
    w = s * 2 + c
    pltpu.sync_copy(zeros, acc.at[pl.ds(s * ZR, ZR)])
    pltpu.sync_copy(srcp.at[w], src_v)
    pltpu.sync_copy(dstp.at[w], dst_v)
    plsc.subcore_barrier()

    def step(b, carry):
      pltpu.sync_copy(h.at[src_v.at[b]], rows_v)
      pltpu.sync_copy(rows_v, acc.at[dst_v.at[b]], add=True)
      return carry

    lax.fori_loop(0, NB, step, 0)
    plsc.subcore_barrier()
    pltpu.sync_copy(acc.at[pl.ds(s * ZR, ZR)],
                    out.at[pl.ds(c * ACC_ROWS + s * ZR, ZR)])

  return k


def _combine_call(aggp, h, degp, wl, bl, wr, wln):
  N, fin = h.shape
  fout = wl.shape[1]
  has_ln = wln is not None
  blk = 400
  grid = (N // blk,)

  def body(agg_ref, h_ref, deg_ref, wl_ref, bl_ref, wr_ref, *rest):
    if has_ln:
      wln_ref, out_ref = rest
    else:
      (out_ref,) = rest
    d = deg_ref[0, :, 0:1] + deg_ref[1, :, 0:1]
    inv = 1.0 / jnp.maximum(d, 1.0)
    a = (agg_ref[0] + agg_ref[1]) * inv
    hv = h_ref[...]
    y = (jnp.dot(a, wl_ref[...], preferred_element_type=jnp.float32)
         + bl_ref[...]
         + jnp.dot(hv, wr_ref[...], preferred_element_type=jnp.float32))
    y = jnp.maximum(y, 0.0)
    if has_ln:
      y = jnp.maximum(
          jnp.dot(y, wln_ref[...], preferred_element_type=jnp.float32) + hv,
          0.0)
    out_ref[...] = y

  in_specs = [
      pl.BlockSpec((2, blk, fin), lambda i: (0, i, 0)),
      pl.BlockSpec((blk, fin), lambda i: (i, 0)),
      pl.BlockSpec((2, blk, 128), lambda i: (0, i, 0)),
      pl.BlockSpec((fin, fout), lambda i: (0, 0)),
      pl.BlockSpec((1, fout), lambda i: (0, 0)),
      pl.BlockSpec((fin, fout), lambda i: (0, 0)),
  ]
  args = [aggp, h, degp, wl, bl.reshape(1, fout), wr]
  if has_ln:
    in_specs.append(pl.BlockSpec((fout, fout), lambda i: (0, 0)))
    args.append(wln)

  return pl.pallas_call(
      body,
      grid=grid,
      in_specs=in_specs,
      out_specs=pl.BlockSpec((blk, fout), lambda i: (i, 0)),
      out_shape=jax.ShapeDtypeStruct((N, fout), jnp.float32),
  )(*args)


def _pool_mlp_call(h, batch, wg1, bg1, wg2, bg2, wg3, bg3, B):
  N, F = h.shape
  CH = 64

  def body(h_ref, b_ref, w1_ref, b1_ref, w2_ref, b2_ref, w3_ref, b3_ref,
           out_ref, pooled_ref):
    brow = b_ref[0, :]

    def graph_body(g, carry):
      start = jnp.sum((brow < g).astype(jnp.int32))
      end = jnp.sum((brow <= g).astype(jnp.int32))

      def cond(c):
        k, _ = c
        return k * CH < end

      def chunk(c):
        k, acc = c
        base = pl.multiple_of(jnp.minimum(k * CH, N - CH), 8)
        rows = h_ref[pl.ds(base, CH), :]
        ridx = base + lax.broadcasted_iota(jnp.int32, (CH, 1), 0)
        m = (ridx >= start) & (ridx < end)
        rows = jnp.where(m, rows, -jnp.inf)
        acc = jnp.maximum(acc, jnp.max(rows, axis=0, keepdims=True))
        return k + 1, acc

      _, acc = lax.while_loop(cond, chunk,
                              (start // CH,
                               jnp.full((1, F), -jnp.inf, jnp.float32)))
      acc = jnp.where(jnp.isfinite(acc), acc, 0.0)
      pooled_ref[pl.ds(g, 1), :] = acc
      return carry

    lax.fori_loop(0, B, graph_body, 0)
    p = pooled_ref[...]
    g1 = jnp.maximum(
        jnp.dot(p, w1_ref[...], preferred_element_type=jnp.float32)
        + b1_ref[...], 0.0)
    g2 = jnp.dot(g1, w2_ref[...], preferred_element_type=jnp.float32) + b2_ref[...]
    out_ref[...] = jnp.maximum(
        jnp.dot(g2, w3_ref[...], preferred_element_type=jnp.float32)
        + b3_ref[...], 0.0)

  return pl.pallas_call(
      body,
      out_shape=jax.ShapeDtypeStruct((B, wg3.shape[1]), jnp.float32),
      scratch_shapes=[pltpu.VMEM((B, F), jnp.float32)],
  )(h, batch.reshape(1, N), wg1, bg1.reshape(1, -1), wg2, bg2.reshape(1, -1),
    wg3, bg3.reshape(1, -1))


def kernel(x, edge_index, batch, Wl1, bl1, Wr1, Wl2, bl2, Wr2,
           Wls, bls, Wrs, Wln, Wg1, bg1, Wg2, bg2, Wg3, bg3):
  N, _ = x.shape
  E = edge_index.shape[1]
  B = 256
  src = edge_index[0]
  dst = edge_index[1]

  NB = -(-E // (NSHARD * EB))
  EP = NSHARD * NB * EB
  pad_src = jnp.arange(EP - E, dtype=jnp.int32) % N
  srcp = jnp.concatenate([src, pad_src]).reshape(NSHARD, NB, EB)
  pad_dst = N + jnp.arange(EP - E, dtype=jnp.int32) % (ACC_ROWS - N)
  dstp = jnp.concatenate([dst, pad_dst]).reshape(NSHARD, NB, EB)
  zeros = jnp.zeros((ZR, FC), jnp.float32)

  aggk = _agg_kernel(N, NB)

  def parts(hcols):
    out = aggk(hcols, srcp, dstp, zeros)
    return out.reshape(2, ACC_ROWS, FC)[:, :N, :]

  def aggregate(h):
    if h.shape[1] == FC:
      return parts(h)
    halves = [parts(h[:, i * FC:(i + 1) * FC]) for i in range(h.shape[1] // FC)]
    return jnp.concatenate(halves, axis=2)

  degp = parts(jnp.ones((N, FC), jnp.float32))

  h = _combine_call(aggregate(x), x, degp, Wl1, bl1, Wr1, None)
  h = _combine_call(aggregate(h), h, degp, Wl2, bl2, Wr2, None)
  for _ in range(4):
    h = _combine_call(aggregate(h), h, degp, Wls, bls, Wrs, Wln)

  return _pool_mlp_call(h, batch, Wg1, bg1, Wg2, bg2, Wg3, bg3, B)


# speedup vs baseline: 3.0321x; 1.0663x over previous
import functools

import jax
import jax.numpy as jnp
from jax import lax
from jax.experimental import pallas as pl
from jax.experimental.pallas import tpu as pltpu
from jax.experimental.pallas import tpu_sc as plsc

EB = 128
NTILES = 16
NSHARD = 32
FC = 128
PH = 8
ACC_ROWS = 10240
ZR = ACC_ROWS // NTILES


def _agg_kernel(N, NB):
  mesh = plsc.VectorSubcoreMesh(core_axis_name="c", subcore_axis_name="s")

  @functools.partial(
      pl.kernel,
      out_type=jax.ShapeDtypeStruct((2 * ACC_ROWS, FC), jnp.float32),
      mesh=mesh,
      scratch_types=[
          pltpu.VMEM((PH, EB), jnp.int32),
          pltpu.VMEM((PH, EB), jnp.int32),
          pltpu.VMEM((2, EB, FC), jnp.float32),
          pltpu.VMEM_SHARED((ACC_ROWS, FC), jnp.float32),
          pltpu.SemaphoreType.DMA,
          pltpu.SemaphoreType.DMA,
      ],
  )
  def k(h, srcp, dstp, zeros, out, src_v, dst_v, rows_v, acc, s0, s1):
    c = lax.axis_index("c")
    s = lax.axis_index("s")
    w = s * 2 + c
    pltpu.sync_copy(zeros, acc.at[pl.ds(s * ZR, ZR)])
    plsc.subcore_barrier()

    def phase(p, carry):
      pltpu.sync_copy(srcp.at[w, pl.ds(p * PH, PH)], src_v)
      pltpu.sync_copy(dstp.at[w, pl.ds(p * PH, PH)], dst_v)

      def pair(q, inner):
        b0 = 2 * q
        b1 = 2 * q + 1
        d0 = pltpu.async_copy(h.at[src_v.at[b0]], rows_v.at[0], s0)
        d1 = pltpu.async_copy(h.at[src_v.at[b1]], rows_v.at[1], s1)
        d0.wait()
        pltpu.sync_copy(rows_v.at[0], acc.at[dst_v.at[b0]], add=True)
        d1.wait()
        pltpu.sync_copy(rows_v.at[1], acc.at[dst_v.at[b1]], add=True)
        return inner

      lax.fori_loop(0, PH // 2, pair, 0)
      return carry

    lax.fori_loop(0, NB // PH, phase, 0)
    plsc.subcore_barrier()
    pltpu.sync_copy(acc.at[pl.ds(s * ZR, ZR)],
                    out.at[pl.ds(c * ACC_ROWS + s * ZR, ZR)])

  return k


def _combine_call(aggp, h, degp, wl, bl, wr, wln):
  N, fin = h.shape
  fout = wl.shape[1]
  has_ln = wln is not None
  blk = 400
  grid = (N // blk,)

  def body(agg_ref, h_ref, deg_ref, wl_ref, bl_ref, wr_ref, *rest):
    if has_ln:
      wln_ref, out_ref = rest
    else:
      (out_ref,) = rest
    d = deg_ref[0, :, 0:1] + deg_ref[1, :, 0:1]
    inv = 1.0 / jnp.maximum(d, 1.0)
    a = (agg_ref[0] + agg_ref[1]) * inv
    hv = h_ref[...]
    y = (jnp.dot(a, wl_ref[...], preferred_element_type=jnp.float32)
         + bl_ref[...]
         + jnp.dot(hv, wr_ref[...], preferred_element_type=jnp.float32))
    y = jnp.maximum(y, 0.0)
    if has_ln:
      y = jnp.maximum(
          jnp.dot(y, wln_ref[...], preferred_element_type=jnp.float32) + hv,
          0.0)
    out_ref[...] = y

  in_specs = [
      pl.BlockSpec((2, blk, fin), lambda i: (0, i, 0)),
      pl.BlockSpec((blk, fin), lambda i: (i, 0)),
      pl.BlockSpec((2, blk, 128), lambda i: (0, i, 0)),
      pl.BlockSpec((fin, fout), lambda i: (0, 0)),
      pl.BlockSpec((1, fout), lambda i: (0, 0)),
      pl.BlockSpec((fin, fout), lambda i: (0, 0)),
  ]
  args = [aggp, h, degp, wl, bl.reshape(1, fout), wr]
  if has_ln:
    in_specs.append(pl.BlockSpec((fout, fout), lambda i: (0, 0)))
    args.append(wln)

  return pl.pallas_call(
      body,
      grid=grid,
      in_specs=in_specs,
      out_specs=pl.BlockSpec((blk, fout), lambda i: (i, 0)),
      out_shape=jax.ShapeDtypeStruct((N, fout), jnp.float32),
  )(*args)


def _pool_mlp_call(h, batch, wg1, bg1, wg2, bg2, wg3, bg3, B):
  N, F = h.shape
  CH = 64

  def body(h_ref, b_ref, w1_ref, b1_ref, w2_ref, b2_ref, w3_ref, b3_ref,
           out_ref, pooled_ref):
    brow = b_ref[0, :]

    def graph_body(g, carry):
      start = jnp.sum((brow < g).astype(jnp.int32))
      end = jnp.sum((brow <= g).astype(jnp.int32))

      def cond(c):
        k, _ = c
        return k * CH < end

      def chunk(c):
        k, acc = c
        base = pl.multiple_of(jnp.minimum(k * CH, N - CH), 8)
        rows = h_ref[pl.ds(base, CH), :]
        ridx = base + lax.broadcasted_iota(jnp.int32, (CH, 1), 0)
        m = (ridx >= start) & (ridx < end)
        rows = jnp.where(m, rows, -jnp.inf)
        acc = jnp.maximum(acc, jnp.max(rows, axis=0, keepdims=True))
        return k + 1, acc

      _, acc = lax.while_loop(cond, chunk,
                              (start // CH,
                               jnp.full((1, F), -jnp.inf, jnp.float32)))
      acc = jnp.where(jnp.isfinite(acc), acc, 0.0)
      pooled_ref[pl.ds(g, 1), :] = acc
      return carry

    lax.fori_loop(0, B, graph_body, 0)
    p = pooled_ref[...]
    g1 = jnp.maximum(
        jnp.dot(p, w1_ref[...], preferred_element_type=jnp.float32)
        + b1_ref[...], 0.0)
    g2 = jnp.dot(g1, w2_ref[...], preferred_element_type=jnp.float32) + b2_ref[...]
    out_ref[...] = jnp.maximum(
        jnp.dot(g2, w3_ref[...], preferred_element_type=jnp.float32)
        + b3_ref[...], 0.0)

  return pl.pallas_call(
      body,
      out_shape=jax.ShapeDtypeStruct((B, wg3.shape[1]), jnp.float32),
      scratch_shapes=[pltpu.VMEM((B, F), jnp.float32)],
  )(h, batch.reshape(1, N), wg1, bg1.reshape(1, -1), wg2, bg2.reshape(1, -1),
    wg3, bg3.reshape(1, -1))


def kernel(x, edge_index, batch, Wl1, bl1, Wr1, Wl2, bl2, Wr2,
           Wls, bls, Wrs, Wln, Wg1, bg1, Wg2, bg2, Wg3, bg3):
  N, _ = x.shape
  E = edge_index.shape[1]
  B = 256
  src = edge_index[0]
  dst = edge_index[1]

  NB = PH * (-(-E // (NSHARD * EB * PH)))
  EP = NSHARD * NB * EB
  pad_src = jnp.arange(EP - E, dtype=jnp.int32) % N
  srcp = jnp.concatenate([src, pad_src]).reshape(NSHARD, NB, EB)
  pad_dst = N + jnp.arange(EP - E, dtype=jnp.int32) % (ACC_ROWS - N)
  dstp = jnp.concatenate([dst, pad_dst]).reshape(NSHARD, NB, EB)
  zeros = jnp.zeros((ZR, FC), jnp.float32)

  aggk = _agg_kernel(N, NB)

  def parts(hcols):
    out = aggk(hcols, srcp, dstp, zeros)
    return out.reshape(2, ACC_ROWS, FC)[:, :N, :]

  def aggregate(h):
    if h.shape[1] == FC:
      return parts(h)
    halves = [parts(h[:, i * FC:(i + 1) * FC]) for i in range(h.shape[1] // FC)]
    return jnp.concatenate(halves, axis=2)

  degp = parts(jnp.ones((N, FC), jnp.float32))

  h = _combine_call(aggregate(x), x, degp, Wl1, bl1, Wr1, None)
  h = _combine_call(aggregate(h), h, degp, Wl2, bl2, Wr2, None)
  for _ in range(4):
    h = _combine_call(aggregate(h), h, degp, Wls, bls, Wrs, Wln)

  return _pool_mlp_call(h, batch, Wg1, bg1, Wg2, bg2, Wg3, bg3, B)
